# Initial kernel scaffold; baseline (speedup 1.0000x reference)
import jax
import jax.numpy as jnp
from jax.experimental import pallas as pl


def kernel(embeddings, edge_index, sub_pos, sub_neg, raw_alpha):
    raise NotImplementedError("write your pallas kernel here")


# speedup vs baseline: 5.6583x; 5.6583x over previous
import jax
import jax.numpy as jnp
from jax import lax
from jax.experimental import pallas as pl
from jax.experimental.pallas import tpu as pltpu
from jax.experimental.pallas import tpu_sc as plsc

N_NODES = 10000
NPAD = 10240
T = 16
ROWS_PER_TILE = NPAD // T
CHUNK = 128
CHUNKS_PER_TILE = 157
EDGES_PAD = T * CHUNKS_PER_TILE * CHUNK
KS = 5
NSUB = 1000
SUB_PAD = 1024
VCHUNKS = ROWS_PER_TILE // 16
LN2 = 0.6931471805599453


def _ln(x):
    xi = plsc.bitcast(x, jnp.int32)
    e = lax.shift_right_arithmetic(xi, 23) - 127
    m = plsc.bitcast(
        lax.bitwise_or(lax.bitwise_and(xi, 0x007FFFFF), 0x3F800000), jnp.float32
    )
    s = (m - 1.0) / (m + 1.0)
    s2 = s * s
    poly = 1.0 + s2 * (1.0 / 3.0 + s2 * (1.0 / 5.0 + s2 * (1.0 / 7.0 + s2 * (1.0 / 9.0))))
    return e.astype(jnp.float32) * LN2 + 2.0 * s * poly


def _body(rows_hbm, cols_hbm, pos_hbm, neg_hbm, alpha_hbm,
          e0_hbm, e1_hbm, loss_hbm,
          row_v, col_v, g0, g1, eold0, eold1, nbuf0, nbuf1,
          dinv_v, zb, ones_v, posv, negv, alv, ne0_sh, ne1_sh):
    t = lax.axis_index("s")
    base = t * ROWS_PER_TILE
    my_rows = pl.ds(base, ROWS_PER_TILE)

    pltpu.sync_copy(rows_hbm.at[t], row_v)
    pltpu.sync_copy(cols_hbm.at[t], col_v)
    pltpu.sync_copy(pos_hbm, posv)
    pltpu.sync_copy(neg_hbm, negv)
    pltpu.sync_copy(alpha_hbm, alv)

    zeros16 = jnp.zeros((16,), jnp.float32)
    ones16 = jnp.ones((16,), jnp.float32)
    for i in range(VCHUNKS):
        zb[pl.ds(i * 16, 16)] = zeros16
    for i in range(CHUNK // 16):
        ones_v[pl.ds(i * 16, 16)] = ones16

    pltpu.sync_copy(zb, e0_hbm.at[my_rows])
    pltpu.sync_copy(zb, e1_hbm.at[my_rows])
    pltpu.sync_copy(zb, ne0_sh.at[my_rows])
    plsc.subcore_barrier()

    @pl.when(t < 8)
    def _():
        pltpu.sync_copy(ones_v, e1_hbm.at[posv.at[t]])

    @pl.when(t >= 8)
    def _():
        pltpu.sync_copy(ones_v, e0_hbm.at[negv.at[t - 8]])

    @pl.loop(0, CHUNKS_PER_TILE)
    def _(j):
        pltpu.sync_copy(ones_v, ne0_sh.at[row_v.at[j]], add=True)

    plsc.subcore_barrier()
    pltpu.sync_copy(ne0_sh.at[my_rows], nbuf0)
    for i in range(VCHUNKS):
        sl = pl.ds(i * 16, 16)
        dinv_v[sl] = 1.0 / jnp.maximum(nbuf0[sl], 1e-12)

    a = alv[...]
    alpha = 1.0 / (1.0 + jnp.exp(-a))
    one_m_alpha = 1.0 - alpha
    plsc.subcore_barrier()

    @pl.loop(0, KS)
    def _(s):
        pltpu.sync_copy(zb, ne0_sh.at[my_rows])
        pltpu.sync_copy(zb, ne1_sh.at[my_rows])
        plsc.subcore_barrier()

        @pl.loop(0, CHUNKS_PER_TILE)
        def _(j):
            pltpu.sync_copy(e0_hbm.at[col_v.at[j]], g0)
            pltpu.sync_copy(e1_hbm.at[col_v.at[j]], g1)
            pltpu.sync_copy(g0, ne0_sh.at[row_v.at[j]], add=True)
            pltpu.sync_copy(g1, ne1_sh.at[row_v.at[j]], add=True)

        plsc.subcore_barrier()

        pltpu.sync_copy(ne0_sh.at[my_rows], nbuf0)
        pltpu.sync_copy(ne1_sh.at[my_rows], nbuf1)
        pltpu.sync_copy(e0_hbm.at[my_rows], eold0)
        pltpu.sync_copy(e1_hbm.at[my_rows], eold1)
        for i in range(VCHUNKS):
            sl = pl.ds(i * 16, 16)
            di = dinv_v[sl]
            eold0[sl] = alpha * eold0[sl] + one_m_alpha * di * nbuf0[sl]
            eold1[sl] = alpha * eold1[sl] + one_m_alpha * di * nbuf1[sl]
        pltpu.sync_copy(eold0, e0_hbm.at[my_rows])
        pltpu.sync_copy(eold1, e1_hbm.at[my_rows])
        plsc.subcore_barrier()

    @pl.when(t == 0)
    def _():
        iot = lax.iota(jnp.int32, 16)
        acc = jnp.zeros((16,), jnp.float32)
        for j in range(SUB_PAD // CHUNK):
            pltpu.sync_copy(e1_hbm.at[posv.at[j]], g0)
            pltpu.sync_copy(e0_hbm.at[negv.at[j]], g1)
            for i in range(CHUNK // 16):
                sl = pl.ds(i * 16, 16)
                gidx = j * CHUNK + i * 16 + iot
                mask = gidx < NSUB
                p = jnp.maximum(g0[sl], 1e-6)
                q = jnp.maximum(g1[sl], 1e-6)
                acc = acc + jnp.where(mask, _ln(p) + _ln(q), 0.0)
        total = jnp.sum(acc * (-1.0 / NSUB))
        alv[...] = jnp.broadcast_to(total, (16,))
        pltpu.sync_copy(alv, loss_hbm)


def kernel(embeddings, edge_index, sub_pos, sub_neg, raw_alpha):
    del embeddings
    row = edge_index[0]
    col = edge_index[1]
    pad_e = EDGES_PAD - row.shape[0]
    pad_idx = jnp.full((pad_e,), NPAD - 1, jnp.int32)
    rows = jnp.concatenate([row, pad_idx]).reshape(T, CHUNKS_PER_TILE, CHUNK)
    cols = jnp.concatenate([col, pad_idx]).reshape(T, CHUNKS_PER_TILE, CHUNK)
    pad_s = jnp.full((SUB_PAD - NSUB,), NPAD - 16, jnp.int32)
    pos = jnp.concatenate([sub_pos, pad_s]).reshape(SUB_PAD // CHUNK, CHUNK)
    neg = jnp.concatenate([sub_neg, pad_s]).reshape(SUB_PAD // CHUNK, CHUNK)
    al = jnp.broadcast_to(raw_alpha.astype(jnp.float32), (16,))

    mesh = plsc.VectorSubcoreMesh(
        core_axis_name="c", subcore_axis_name="s", num_cores=1
    )
    f32 = jnp.float32
    fn = pl.kernel(
        _body,
        compiler_params=pltpu.CompilerParams(needs_layout_passes=False),
        out_type=[
            jax.ShapeDtypeStruct((NPAD,), f32),
            jax.ShapeDtypeStruct((NPAD,), f32),
            jax.ShapeDtypeStruct((16,), f32),
        ],
        mesh=mesh,
        scratch_types=[
            pltpu.VMEM((CHUNKS_PER_TILE, CHUNK), jnp.int32),
            pltpu.VMEM((CHUNKS_PER_TILE, CHUNK), jnp.int32),
            pltpu.VMEM((CHUNK,), f32),
            pltpu.VMEM((CHUNK,), f32),
            pltpu.VMEM((ROWS_PER_TILE,), f32),
            pltpu.VMEM((ROWS_PER_TILE,), f32),
            pltpu.VMEM((ROWS_PER_TILE,), f32),
            pltpu.VMEM((ROWS_PER_TILE,), f32),
            pltpu.VMEM((ROWS_PER_TILE,), f32),
            pltpu.VMEM((ROWS_PER_TILE,), f32),
            pltpu.VMEM((CHUNK,), f32),
            pltpu.VMEM((SUB_PAD // CHUNK, CHUNK), jnp.int32),
            pltpu.VMEM((SUB_PAD // CHUNK, CHUNK), jnp.int32),
            pltpu.VMEM((16,), f32),
            pltpu.VMEM_SHARED((NPAD,), f32),
            pltpu.VMEM_SHARED((NPAD,), f32),
        ],
    )
    e0, e1, lv = fn(rows, cols, pos, neg, al)
    E = jnp.stack([e0[:N_NODES], e1[:N_NODES]], axis=1)
    return (lv[0], E)


# speedup vs baseline: 16.8354x; 2.9753x over previous
import jax
import jax.numpy as jnp
from jax import lax
from jax.experimental import pallas as pl
from jax.experimental.pallas import tpu as pltpu
from jax.experimental.pallas import tpu_sc as plsc

N_NODES = 10000
NPAD = 10240
T = 16
ROWS_PER_TILE = NPAD // T
CHUNK = 128
NCH = 158
EDGES_PAD = T * NCH * CHUNK
KS = 5
NSUB = 1000
SUB_PAD = 1024
SUB_CH = SUB_PAD // CHUNK
VCHUNKS = ROWS_PER_TILE // 16
LN2 = 0.6931471805599453


def _ln(x):
    xi = plsc.bitcast(x, jnp.int32)
    e = lax.shift_right_arithmetic(xi, 23) - 127
    m = plsc.bitcast(
        lax.bitwise_or(lax.bitwise_and(xi, 0x007FFFFF), 0x3F800000), jnp.float32
    )
    s = (m - 1.0) / (m + 1.0)
    s2 = s * s
    poly = 1.0 + s2 * (1.0 / 3.0 + s2 * (1.0 / 5.0 + s2 * (1.0 / 7.0 + s2 * (1.0 / 9.0))))
    return e.astype(jnp.float32) * LN2 + 2.0 * s * poly


def _body(rows_hbm, cols_hbm, sub_hbm, alpha_hbm,
          e_hbm, loss_hbm,
          row_v, col_v, g0, g1, eold, nbuf, dinv_v, zb, ones_v,
          subv, alv, ne_sh, sg0, sg1):
    c = lax.axis_index("c")
    t = lax.axis_index("s")
    base = t * ROWS_PER_TILE
    ebase = c * NPAD + base
    my_rows = pl.ds(base, ROWS_PER_TILE)
    my_erows = pl.ds(ebase, ROWS_PER_TILE)

    pltpu.sync_copy(rows_hbm.at[t], row_v)
    pltpu.sync_copy(cols_hbm.at[c * T + t], col_v)
    pltpu.sync_copy(sub_hbm.at[pl.ds(c * SUB_CH, SUB_CH)], subv)
    pltpu.sync_copy(alpha_hbm, alv)

    zeros16 = jnp.zeros((16,), jnp.float32)
    ones16 = jnp.ones((16,), jnp.float32)
    for i in range(VCHUNKS):
        zb[pl.ds(i * 16, 16)] = zeros16
    for i in range(CHUNK // 16):
        ones_v[pl.ds(i * 16, 16)] = ones16

    pltpu.sync_copy(zb, e_hbm.at[my_erows])
    pltpu.sync_copy(zb, ne_sh.at[my_rows])
    plsc.subcore_barrier()

    @pl.when(t < SUB_CH)
    def _():
        pltpu.sync_copy(ones_v, e_hbm.at[subv.at[t]])

    @pl.loop(0, NCH)
    def _(j):
        pltpu.sync_copy(ones_v, ne_sh.at[row_v.at[j]], add=True)

    plsc.subcore_barrier()
    pltpu.sync_copy(ne_sh.at[my_rows], nbuf)
    for i in range(VCHUNKS):
        sl = pl.ds(i * 16, 16)
        dinv_v[sl] = 1.0 / jnp.maximum(nbuf[sl], 1e-12)

    a = alv[...]
    alpha = 1.0 / (1.0 + jnp.exp(-a))
    one_m_alpha = 1.0 - alpha
    plsc.subcore_barrier()

    @pl.loop(0, KS)
    def _(s):
        pltpu.sync_copy(zb, ne_sh.at[my_rows])
        plsc.subcore_barrier()

        pltpu.async_copy(e_hbm.at[col_v.at[0]], g0, sg0)
        pltpu.async_copy(e_hbm.at[col_v.at[1]], g1, sg1)

        @pl.loop(0, NCH, step=2)
        def _(j):
            pltpu.make_async_copy(e_hbm.at[col_v.at[j]], g0, sg0).wait()
            pltpu.sync_copy(g0, ne_sh.at[row_v.at[j]], add=True)

            @pl.when(j + 2 < NCH)
            def _():
                pltpu.async_copy(e_hbm.at[col_v.at[j + 2]], g0, sg0)

            pltpu.make_async_copy(e_hbm.at[col_v.at[j + 1]], g1, sg1).wait()
            pltpu.sync_copy(g1, ne_sh.at[row_v.at[j + 1]], add=True)

            @pl.when(j + 3 < NCH)
            def _():
                pltpu.async_copy(e_hbm.at[col_v.at[j + 3]], g1, sg1)

        plsc.subcore_barrier()

        pltpu.sync_copy(ne_sh.at[my_rows], nbuf)
        pltpu.sync_copy(e_hbm.at[my_erows], eold)
        for i in range(VCHUNKS):
            sl = pl.ds(i * 16, 16)
            eold[sl] = alpha * eold[sl] + one_m_alpha * dinv_v[sl] * nbuf[sl]
        pltpu.sync_copy(eold, e_hbm.at[my_erows])
        plsc.subcore_barrier()

    @pl.when(t == 0)
    def _():
        iot = lax.iota(jnp.int32, 16)
        acc = jnp.zeros((16,), jnp.float32)
        for j in range(SUB_CH):
            pltpu.sync_copy(e_hbm.at[subv.at[j]], g0)
            for i in range(CHUNK // 16):
                gidx = j * CHUNK + i * 16 + iot
                p = jnp.maximum(g0[pl.ds(i * 16, 16)], 1e-6)
                acc = acc + jnp.where(gidx < NSUB, _ln(p), 0.0)
        total = jnp.sum(acc * (-1.0 / NSUB))
        alv[...] = jnp.broadcast_to(total, (16,))
        pltpu.sync_copy(alv, loss_hbm.at[c])


def kernel(embeddings, edge_index, sub_pos, sub_neg, raw_alpha):
    del embeddings
    row = edge_index[0]
    col = edge_index[1]
    pad_e = EDGES_PAD - row.shape[0]
    pad_idx = jnp.full((pad_e,), NPAD - 1, jnp.int32)
    rows = jnp.concatenate([row, pad_idx]).reshape(T, NCH, CHUNK)
    col_p = jnp.concatenate([col, pad_idx])
    cols = jnp.concatenate([col_p, col_p + NPAD]).reshape(2 * T, NCH, CHUNK)
    pad_s = jnp.full((SUB_PAD - NSUB,), NPAD - 16, jnp.int32)
    sub = jnp.concatenate([
        jnp.concatenate([sub_neg, pad_s]),
        jnp.concatenate([sub_pos, pad_s]) + NPAD,
    ]).reshape(2 * SUB_CH, CHUNK)
    al = jnp.broadcast_to(raw_alpha.astype(jnp.float32), (16,))

    mesh = plsc.VectorSubcoreMesh(core_axis_name="c", subcore_axis_name="s")
    f32 = jnp.float32
    fn = pl.kernel(
        _body,
        compiler_params=pltpu.CompilerParams(needs_layout_passes=False),
        out_type=[
            jax.ShapeDtypeStruct((2 * NPAD,), f32),
            jax.ShapeDtypeStruct((2, 16), f32),
        ],
        mesh=mesh,
        scratch_types=[
            pltpu.VMEM((NCH, CHUNK), jnp.int32),
            pltpu.VMEM((NCH, CHUNK), jnp.int32),
            pltpu.VMEM((CHUNK,), f32),
            pltpu.VMEM((CHUNK,), f32),
            pltpu.VMEM((ROWS_PER_TILE,), f32),
            pltpu.VMEM((ROWS_PER_TILE,), f32),
            pltpu.VMEM((ROWS_PER_TILE,), f32),
            pltpu.VMEM((ROWS_PER_TILE,), f32),
            pltpu.VMEM((CHUNK,), f32),
            pltpu.VMEM((SUB_CH, CHUNK), jnp.int32),
            pltpu.VMEM((16,), f32),
            pltpu.VMEM_SHARED((NPAD,), f32),
            pltpu.SemaphoreType.DMA,
            pltpu.SemaphoreType.DMA,
        ],
    )
    e, lv = fn(rows, cols, sub, al)
    E = jnp.stack([e[:N_NODES], e[NPAD:NPAD + N_NODES]], axis=1)
    return (lv[0, 0] + lv[1, 0], E)


# speedup vs baseline: 43.4688x; 2.5820x over previous
import jax
import jax.numpy as jnp
from jax import lax
from jax.experimental import pallas as pl
from jax.experimental.pallas import tpu as pltpu
from jax.experimental.pallas import tpu_sc as plsc

N_NODES = 10000
NPAD = 10240
T = 16
ROWS_PER_TILE = NPAD // T
CHUNK = 128
NCH = 158
EDGES_PAD = T * NCH * CHUNK
KS = 5
NSUB = 1000
SUB_PAD = 1024
SUB_CH = SUB_PAD // CHUNK
VCHUNKS = ROWS_PER_TILE // 16
LN2 = 0.6931471805599453


def _ln(x):
    xi = plsc.bitcast(x, jnp.int32)
    e = lax.shift_right_arithmetic(xi, 23) - 127
    m = plsc.bitcast(
        lax.bitwise_or(lax.bitwise_and(xi, 0x007FFFFF), 0x3F800000), jnp.float32
    )
    s = (m - 1.0) / (m + 1.0)
    s2 = s * s
    poly = 1.0 + s2 * (1.0 / 3.0 + s2 * (1.0 / 5.0 + s2 * (1.0 / 7.0 + s2 * (1.0 / 9.0))))
    return e.astype(jnp.float32) * LN2 + 2.0 * s * poly


def _body(rows_hbm, cols_hbm, sub_hbm, alpha_hbm,
          e_hbm, loss_hbm,
          row_v, col_v, g0, g1, eold, nbuf, dinv_v, zb, ones_v,
          subv, alv, ne_sh, e_sh, sg0, sg1):
    c = lax.axis_index("c")
    t = lax.axis_index("s")
    base = t * ROWS_PER_TILE
    ebase = c * NPAD + base
    my_rows = pl.ds(base, ROWS_PER_TILE)
    my_erows = pl.ds(ebase, ROWS_PER_TILE)

    pltpu.sync_copy(rows_hbm.at[t], row_v)
    pltpu.sync_copy(cols_hbm.at[t], col_v)
    pltpu.sync_copy(sub_hbm.at[pl.ds(c * SUB_CH, SUB_CH)], subv)
    pltpu.sync_copy(alpha_hbm, alv)

    zeros16 = jnp.zeros((16,), jnp.float32)
    ones16 = jnp.ones((16,), jnp.float32)
    for i in range(VCHUNKS):
        zb[pl.ds(i * 16, 16)] = zeros16
    for i in range(CHUNK // 16):
        ones_v[pl.ds(i * 16, 16)] = ones16

    pltpu.sync_copy(zb, e_sh.at[my_rows])
    pltpu.sync_copy(zb, ne_sh.at[my_rows])
    plsc.subcore_barrier()

    @pl.when(t < SUB_CH)
    def _():
        pltpu.sync_copy(ones_v, e_sh.at[subv.at[t]])

    @pl.loop(0, NCH)
    def _(j):
        pltpu.sync_copy(ones_v, ne_sh.at[row_v.at[j]], add=True)

    plsc.subcore_barrier()
    pltpu.sync_copy(ne_sh.at[my_rows], nbuf)
    for i in range(VCHUNKS):
        sl = pl.ds(i * 16, 16)
        dinv_v[sl] = 1.0 / jnp.maximum(nbuf[sl], 1e-12)

    a = alv[...]
    alpha = 1.0 / (1.0 + jnp.exp(-a))
    one_m_alpha = 1.0 - alpha
    plsc.subcore_barrier()

    @pl.loop(0, KS)
    def _(s):
        pltpu.sync_copy(zb, ne_sh.at[my_rows])
        plsc.subcore_barrier()

        pltpu.async_copy(e_sh.at[col_v.at[0]], g0, sg0)
        pltpu.async_copy(e_sh.at[col_v.at[1]], g1, sg1)

        @pl.loop(0, NCH, step=2)
        def _(j):
            pltpu.make_async_copy(e_sh.at[col_v.at[j]], g0, sg0).wait()
            pltpu.sync_copy(g0, ne_sh.at[row_v.at[j]], add=True)

            @pl.when(j + 2 < NCH)
            def _():
                pltpu.async_copy(e_sh.at[col_v.at[j + 2]], g0, sg0)

            pltpu.make_async_copy(e_sh.at[col_v.at[j + 1]], g1, sg1).wait()
            pltpu.sync_copy(g1, ne_sh.at[row_v.at[j + 1]], add=True)

            @pl.when(j + 3 < NCH)
            def _():
                pltpu.async_copy(e_sh.at[col_v.at[j + 3]], g1, sg1)

        plsc.subcore_barrier()

        pltpu.sync_copy(ne_sh.at[my_rows], nbuf)
        pltpu.sync_copy(e_sh.at[my_rows], eold)
        for i in range(VCHUNKS):
            sl = pl.ds(i * 16, 16)
            eold[sl] = alpha * eold[sl] + one_m_alpha * dinv_v[sl] * nbuf[sl]
        pltpu.sync_copy(eold, e_sh.at[my_rows])

        @pl.when(s == KS - 1)
        def _():
            pltpu.sync_copy(eold, e_hbm.at[my_erows])

        plsc.subcore_barrier()

    @pl.when(t == 0)
    def _():
        iot = lax.iota(jnp.int32, 16)
        acc = jnp.zeros((16,), jnp.float32)
        for j in range(SUB_CH):
            pltpu.sync_copy(e_sh.at[subv.at[j]], g0)
            for i in range(CHUNK // 16):
                gidx = j * CHUNK + i * 16 + iot
                p = jnp.maximum(g0[pl.ds(i * 16, 16)], 1e-6)
                acc = acc + jnp.where(gidx < NSUB, _ln(p), 0.0)
        total = jnp.sum(acc * (-1.0 / NSUB))
        alv[...] = jnp.broadcast_to(total, (16,))
        pltpu.sync_copy(alv, loss_hbm.at[c])


def kernel(embeddings, edge_index, sub_pos, sub_neg, raw_alpha):
    del embeddings
    row = edge_index[0]
    col = edge_index[1]
    pad_e = EDGES_PAD - row.shape[0]
    pad_idx = jnp.full((pad_e,), NPAD - 1, jnp.int32)
    rows = jnp.concatenate([row, pad_idx]).reshape(T, NCH, CHUNK)
    cols = jnp.concatenate([col, pad_idx]).reshape(T, NCH, CHUNK)
    pad_s = jnp.full((SUB_PAD - NSUB,), NPAD - 16, jnp.int32)
    sub = jnp.concatenate([
        jnp.concatenate([sub_neg, pad_s]),
        jnp.concatenate([sub_pos, pad_s]),
    ]).reshape(2 * SUB_CH, CHUNK)
    al = jnp.broadcast_to(raw_alpha.astype(jnp.float32), (16,))

    mesh = plsc.VectorSubcoreMesh(core_axis_name="c", subcore_axis_name="s")
    f32 = jnp.float32
    fn = pl.kernel(
        _body,
        compiler_params=pltpu.CompilerParams(needs_layout_passes=False),
        out_type=[
            jax.ShapeDtypeStruct((2 * NPAD,), f32),
            jax.ShapeDtypeStruct((2, 16), f32),
        ],
        mesh=mesh,
        scratch_types=[
            pltpu.VMEM((NCH, CHUNK), jnp.int32),
            pltpu.VMEM((NCH, CHUNK), jnp.int32),
            pltpu.VMEM((CHUNK,), f32),
            pltpu.VMEM((CHUNK,), f32),
            pltpu.VMEM((ROWS_PER_TILE,), f32),
            pltpu.VMEM((ROWS_PER_TILE,), f32),
            pltpu.VMEM((ROWS_PER_TILE,), f32),
            pltpu.VMEM((ROWS_PER_TILE,), f32),
            pltpu.VMEM((CHUNK,), f32),
            pltpu.VMEM((SUB_CH, CHUNK), jnp.int32),
            pltpu.VMEM((16,), f32),
            pltpu.VMEM_SHARED((NPAD,), f32),
            pltpu.VMEM_SHARED((NPAD,), f32),
            pltpu.SemaphoreType.DMA,
            pltpu.SemaphoreType.DMA,
        ],
    )
    e, lv = fn(rows, cols, sub, al)
    E = jnp.stack([e[:N_NODES], e[NPAD:NPAD + N_NODES]], axis=1)
    return (lv[0, 0] + lv[1, 0], E)


# speedup vs baseline: 46.4490x; 1.0686x over previous
import jax
import jax.numpy as jnp
from jax import lax
from jax.experimental import pallas as pl
from jax.experimental.pallas import tpu as pltpu
from jax.experimental.pallas import tpu_sc as plsc

N_NODES = 10000
NPAD = 10240
T = 16
ROWS_PER_TILE = NPAD // T
CHUNK = 128
NCH = 160
EDGES_PAD = T * NCH * CHUNK
KS = 5
NSUB = 1000
SUB_PAD = 1024
SUB_CH = SUB_PAD // CHUNK
VCHUNKS = ROWS_PER_TILE // 16
LN2 = 0.6931471805599453


def _ln(x):
    xi = plsc.bitcast(x, jnp.int32)
    e = lax.shift_right_arithmetic(xi, 23) - 127
    m = plsc.bitcast(
        lax.bitwise_or(lax.bitwise_and(xi, 0x007FFFFF), 0x3F800000), jnp.float32
    )
    s = (m - 1.0) / (m + 1.0)
    s2 = s * s
    poly = 1.0 + s2 * (1.0 / 3.0 + s2 * (1.0 / 5.0 + s2 * (1.0 / 7.0 + s2 * (1.0 / 9.0))))
    return e.astype(jnp.float32) * LN2 + 2.0 * s * poly


def _body(rows_hbm, cols_hbm, sub_hbm, alpha_hbm,
          e_hbm, loss_hbm,
          row_v, col_v, g0, g1, g2, g3, eold, nbuf, dinv_v, zb, ones_v,
          subv, alv, ne_sh, e_sh, sg0, sg1, sg2, sg3, ss0, ss1, ss2, ss3):
    c = lax.axis_index("c")
    t = lax.axis_index("s")
    base = t * ROWS_PER_TILE
    ebase = c * NPAD + base
    my_rows = pl.ds(base, ROWS_PER_TILE)
    my_erows = pl.ds(ebase, ROWS_PER_TILE)

    pltpu.sync_copy(rows_hbm.at[t], row_v)
    pltpu.sync_copy(cols_hbm.at[t], col_v)
    pltpu.sync_copy(sub_hbm.at[pl.ds(c * SUB_CH, SUB_CH)], subv)
    pltpu.sync_copy(alpha_hbm, alv)

    zeros16 = jnp.zeros((16,), jnp.float32)
    ones16 = jnp.ones((16,), jnp.float32)
    for i in range(VCHUNKS):
        zb[pl.ds(i * 16, 16)] = zeros16
    for i in range(CHUNK // 16):
        ones_v[pl.ds(i * 16, 16)] = ones16

    pltpu.sync_copy(zb, e_sh.at[my_rows])
    pltpu.sync_copy(zb, ne_sh.at[my_rows])
    plsc.subcore_barrier()

    @pl.when(t < SUB_CH)
    def _():
        pltpu.sync_copy(ones_v, e_sh.at[subv.at[t]])

    @pl.loop(0, NCH)
    def _(j):
        pltpu.async_copy(ones_v, ne_sh.at[row_v.at[j]], ss0, add=True)

    @pl.loop(0, NCH)
    def _(j):
        pltpu.make_async_copy(ones_v, ne_sh.at[row_v.at[j]], ss0).wait()

    plsc.subcore_barrier()
    pltpu.sync_copy(ne_sh.at[my_rows], nbuf)
    for i in range(VCHUNKS):
        sl = pl.ds(i * 16, 16)
        dinv_v[sl] = 1.0 / jnp.maximum(nbuf[sl], 1e-12)

    a = alv[...]
    alpha = 1.0 / (1.0 + jnp.exp(-a))
    one_m_alpha = 1.0 - alpha
    plsc.subcore_barrier()

    @pl.loop(0, KS)
    def _(s):
        pltpu.sync_copy(zb, ne_sh.at[my_rows])
        plsc.subcore_barrier()

        gbufs = (g0, g1, g2, g3)
        gsems = (sg0, sg1, sg2, sg3)
        ssems = (ss0, ss1, ss2, ss3)
        for b in range(4):
            pltpu.async_copy(e_sh.at[col_v.at[b]], gbufs[b], gsems[b])

        @pl.loop(0, NCH, step=4)
        def _(j):
            for b in range(4):
                pltpu.make_async_copy(
                    e_sh.at[col_v.at[j + b]], gbufs[b], gsems[b]).wait()
                pltpu.async_copy(
                    gbufs[b], ne_sh.at[row_v.at[j + b]], ssems[b], add=True)
            for b in range(4):
                @pl.when(j + b + 4 < NCH)
                def _(b=b):
                    pltpu.make_async_copy(
                        gbufs[b], ne_sh.at[row_v.at[j + b]], ssems[b]).wait()
                    pltpu.async_copy(
                        e_sh.at[col_v.at[j + b + 4]], gbufs[b], gsems[b])

        for b in range(4):
            pltpu.make_async_copy(
                gbufs[b], ne_sh.at[row_v.at[NCH - 4 + b]], ssems[b]).wait()

        plsc.subcore_barrier()

        nd = pltpu.async_copy(ne_sh.at[my_rows], nbuf, sg0)
        ed = pltpu.async_copy(e_sh.at[my_rows], eold, sg1)
        nd.wait()
        ed.wait()
        for i in range(VCHUNKS):
            sl = pl.ds(i * 16, 16)
            eold[sl] = alpha * eold[sl] + one_m_alpha * dinv_v[sl] * nbuf[sl]
        pltpu.sync_copy(eold, e_sh.at[my_rows])

        @pl.when(s == KS - 1)
        def _():
            pltpu.sync_copy(eold, e_hbm.at[my_erows])

        plsc.subcore_barrier()

    @pl.when(t == 0)
    def _():
        iot = lax.iota(jnp.int32, 16)
        acc = jnp.zeros((16,), jnp.float32)
        for j in range(SUB_CH):
            pltpu.sync_copy(e_sh.at[subv.at[j]], g0)
            for i in range(CHUNK // 16):
                gidx = j * CHUNK + i * 16 + iot
                p = jnp.maximum(g0[pl.ds(i * 16, 16)], 1e-6)
                acc = acc + jnp.where(gidx < NSUB, _ln(p), 0.0)
        total = jnp.sum(acc * (-1.0 / NSUB))
        alv[...] = jnp.broadcast_to(total, (16,))
        pltpu.sync_copy(alv, loss_hbm.at[c])


def kernel(embeddings, edge_index, sub_pos, sub_neg, raw_alpha):
    del embeddings
    row = edge_index[0]
    col = edge_index[1]
    pad_e = EDGES_PAD - row.shape[0]
    pad_idx = jnp.full((pad_e,), NPAD - 1, jnp.int32)
    rows = jnp.concatenate([row, pad_idx]).reshape(T, NCH, CHUNK)
    cols = jnp.concatenate([col, pad_idx]).reshape(T, NCH, CHUNK)
    pad_s = jnp.full((SUB_PAD - NSUB,), NPAD - 16, jnp.int32)
    sub = jnp.concatenate([
        jnp.concatenate([sub_neg, pad_s]),
        jnp.concatenate([sub_pos, pad_s]),
    ]).reshape(2 * SUB_CH, CHUNK)
    al = jnp.broadcast_to(raw_alpha.astype(jnp.float32), (16,))

    mesh = plsc.VectorSubcoreMesh(core_axis_name="c", subcore_axis_name="s")
    f32 = jnp.float32
    fn = pl.kernel(
        _body,
        compiler_params=pltpu.CompilerParams(needs_layout_passes=False),
        out_type=[
            jax.ShapeDtypeStruct((2 * NPAD,), f32),
            jax.ShapeDtypeStruct((2, 16), f32),
        ],
        mesh=mesh,
        scratch_types=[
            pltpu.VMEM((NCH, CHUNK), jnp.int32),
            pltpu.VMEM((NCH, CHUNK), jnp.int32),
            pltpu.VMEM((CHUNK,), f32),
            pltpu.VMEM((CHUNK,), f32),
            pltpu.VMEM((CHUNK,), f32),
            pltpu.VMEM((CHUNK,), f32),
            pltpu.VMEM((ROWS_PER_TILE,), f32),
            pltpu.VMEM((ROWS_PER_TILE,), f32),
            pltpu.VMEM((ROWS_PER_TILE,), f32),
            pltpu.VMEM((ROWS_PER_TILE,), f32),
            pltpu.VMEM((CHUNK,), f32),
            pltpu.VMEM((SUB_CH, CHUNK), jnp.int32),
            pltpu.VMEM((16,), f32),
            pltpu.VMEM_SHARED((NPAD,), f32),
            pltpu.VMEM_SHARED((NPAD,), f32),
            pltpu.SemaphoreType.DMA,
            pltpu.SemaphoreType.DMA,
            pltpu.SemaphoreType.DMA,
            pltpu.SemaphoreType.DMA,
            pltpu.SemaphoreType.DMA,
            pltpu.SemaphoreType.DMA,
            pltpu.SemaphoreType.DMA,
            pltpu.SemaphoreType.DMA,
        ],
    )
    e, lv = fn(rows, cols, sub, al)
    E = jnp.stack([e[:N_NODES], e[NPAD:NPAD + N_NODES]], axis=1)
    return (lv[0, 0] + lv[1, 0], E)


# speedup vs baseline: 47.0589x; 1.0131x over previous
import jax
import jax.numpy as jnp
from jax import lax
from jax.experimental import pallas as pl
from jax.experimental.pallas import tpu as pltpu
from jax.experimental.pallas import tpu_sc as plsc

N_NODES = 10000
NPAD = 10240
T = 16
ROWS_PER_TILE = NPAD // T
CHUNK = 128
NCH = 160
EDGES_PAD = T * NCH * CHUNK
KS = 5
NSUB = 1000
SUB_PAD = 1024
SUB_CH = SUB_PAD // CHUNK
VCHUNKS = ROWS_PER_TILE // 16
LN2 = 0.6931471805599453


def _ln(x):
    xi = plsc.bitcast(x, jnp.int32)
    e = lax.shift_right_arithmetic(xi, 23) - 127
    m = plsc.bitcast(
        lax.bitwise_or(lax.bitwise_and(xi, 0x007FFFFF), 0x3F800000), jnp.float32
    )
    s = (m - 1.0) / (m + 1.0)
    s2 = s * s
    poly = 1.0 + s2 * (1.0 / 3.0 + s2 * (1.0 / 5.0 + s2 * (1.0 / 7.0 + s2 * (1.0 / 9.0))))
    return e.astype(jnp.float32) * LN2 + 2.0 * s * poly


def _body(rows_hbm, cols_hbm, sub_hbm, alpha_hbm,
          e_hbm, loss_hbm,
          row_v, col_v, g0, g1, g2, g3, eold, nbuf, dinv_v, zb, ones_v,
          subv, alv, ne_sh, e_sh, sg0, sg1, sg2, sg3, ss0, ss1, ss2, ss3):
    c = lax.axis_index("c")
    t = lax.axis_index("s")
    base = t * ROWS_PER_TILE
    ebase = c * NPAD + base
    my_rows = pl.ds(base, ROWS_PER_TILE)
    my_erows = pl.ds(ebase, ROWS_PER_TILE)

    d1 = pltpu.async_copy(rows_hbm.at[t], row_v, sg0)
    d2 = pltpu.async_copy(cols_hbm.at[t], col_v, sg1)
    d3 = pltpu.async_copy(sub_hbm.at[pl.ds(c * SUB_CH, SUB_CH)], subv, sg2)
    d4 = pltpu.async_copy(alpha_hbm, alv, sg3)

    zeros16 = jnp.zeros((16,), jnp.float32)
    ones16 = jnp.ones((16,), jnp.float32)
    for i in range(VCHUNKS):
        zb[pl.ds(i * 16, 16)] = zeros16
    for i in range(CHUNK // 16):
        ones_v[pl.ds(i * 16, 16)] = ones16
    d1.wait()
    d2.wait()
    d3.wait()
    d4.wait()

    pltpu.sync_copy(zb, e_sh.at[my_rows])
    pltpu.sync_copy(zb, ne_sh.at[my_rows])
    plsc.subcore_barrier()

    @pl.when(t < SUB_CH)
    def _():
        pltpu.sync_copy(ones_v, e_sh.at[subv.at[t]])

    @pl.loop(0, NCH)
    def _(j):
        pltpu.async_copy(ones_v, ne_sh.at[row_v.at[j]], ss0, add=True)

    @pl.loop(0, NCH)
    def _(j):
        pltpu.make_async_copy(ones_v, ne_sh.at[row_v.at[j]], ss0).wait()

    plsc.subcore_barrier()
    pltpu.sync_copy(ne_sh.at[my_rows], nbuf)
    pltpu.sync_copy(zb, ne_sh.at[my_rows])
    for i in range(VCHUNKS):
        sl = pl.ds(i * 16, 16)
        dinv_v[sl] = 1.0 / jnp.maximum(nbuf[sl], 1e-12)

    a = alv[...]
    alpha = 1.0 / (1.0 + jnp.exp(-a))
    one_m_alpha = 1.0 - alpha
    plsc.subcore_barrier()

    @pl.loop(0, KS)
    def _(s):
        gbufs = (g0, g1, g2, g3)
        gsems = (sg0, sg1, sg2, sg3)
        ssems = (ss0, ss1, ss2, ss3)
        for b in range(4):
            pltpu.async_copy(e_sh.at[col_v.at[b]], gbufs[b], gsems[b])

        @pl.loop(0, NCH, step=4)
        def _(j):
            for b in range(4):
                pltpu.make_async_copy(
                    e_sh.at[col_v.at[j + b]], gbufs[b], gsems[b]).wait()
                pltpu.async_copy(
                    gbufs[b], ne_sh.at[row_v.at[j + b]], ssems[b], add=True)
            for b in range(4):
                @pl.when(j + b + 4 < NCH)
                def _(b=b):
                    pltpu.make_async_copy(
                        gbufs[b], ne_sh.at[row_v.at[j + b]], ssems[b]).wait()
                    pltpu.async_copy(
                        e_sh.at[col_v.at[j + b + 4]], gbufs[b], gsems[b])

        for b in range(4):
            pltpu.make_async_copy(
                gbufs[b], ne_sh.at[row_v.at[NCH - 4 + b]], ssems[b]).wait()

        plsc.subcore_barrier()

        nd = pltpu.async_copy(ne_sh.at[my_rows], nbuf, sg0)
        ed = pltpu.async_copy(e_sh.at[my_rows], eold, sg1)
        nd.wait()
        zd = pltpu.async_copy(zb, ne_sh.at[my_rows], sg2)
        ed.wait()
        for i in range(VCHUNKS):
            sl = pl.ds(i * 16, 16)
            eold[sl] = alpha * eold[sl] + one_m_alpha * dinv_v[sl] * nbuf[sl]
        pltpu.sync_copy(eold, e_sh.at[my_rows])
        zd.wait()

        @pl.when(s == KS - 1)
        def _():
            pltpu.sync_copy(eold, e_hbm.at[my_erows])

        plsc.subcore_barrier()

    @pl.when(t == 0)
    def _():
        iot = lax.iota(jnp.int32, 16)
        acc = jnp.zeros((16,), jnp.float32)
        for j in range(SUB_CH):
            pltpu.sync_copy(e_sh.at[subv.at[j]], g0)
            for i in range(CHUNK // 16):
                gidx = j * CHUNK + i * 16 + iot
                p = jnp.maximum(g0[pl.ds(i * 16, 16)], 1e-6)
                acc = acc + jnp.where(gidx < NSUB, _ln(p), 0.0)
        total = jnp.sum(acc * (-1.0 / NSUB))
        alv[...] = jnp.broadcast_to(total, (16,))
        pltpu.sync_copy(alv, loss_hbm.at[c])


def kernel(embeddings, edge_index, sub_pos, sub_neg, raw_alpha):
    del embeddings
    row = edge_index[0]
    col = edge_index[1]
    pad_e = EDGES_PAD - row.shape[0]
    pad_idx = jnp.full((pad_e,), NPAD - 1, jnp.int32)
    rows = jnp.concatenate([row, pad_idx]).reshape(T, NCH, CHUNK)
    cols = jnp.concatenate([col, pad_idx]).reshape(T, NCH, CHUNK)
    pad_s = jnp.full((SUB_PAD - NSUB,), NPAD - 16, jnp.int32)
    sub = jnp.concatenate([
        jnp.concatenate([sub_neg, pad_s]),
        jnp.concatenate([sub_pos, pad_s]),
    ]).reshape(2 * SUB_CH, CHUNK)
    al = jnp.broadcast_to(raw_alpha.astype(jnp.float32), (16,))

    mesh = plsc.VectorSubcoreMesh(core_axis_name="c", subcore_axis_name="s")
    f32 = jnp.float32
    fn = pl.kernel(
        _body,
        compiler_params=pltpu.CompilerParams(needs_layout_passes=False),
        out_type=[
            jax.ShapeDtypeStruct((2 * NPAD,), f32),
            jax.ShapeDtypeStruct((2, 16), f32),
        ],
        mesh=mesh,
        scratch_types=[
            pltpu.VMEM((NCH, CHUNK), jnp.int32),
            pltpu.VMEM((NCH, CHUNK), jnp.int32),
            pltpu.VMEM((CHUNK,), f32),
            pltpu.VMEM((CHUNK,), f32),
            pltpu.VMEM((CHUNK,), f32),
            pltpu.VMEM((CHUNK,), f32),
            pltpu.VMEM((ROWS_PER_TILE,), f32),
            pltpu.VMEM((ROWS_PER_TILE,), f32),
            pltpu.VMEM((ROWS_PER_TILE,), f32),
            pltpu.VMEM((ROWS_PER_TILE,), f32),
            pltpu.VMEM((CHUNK,), f32),
            pltpu.VMEM((SUB_CH, CHUNK), jnp.int32),
            pltpu.VMEM((16,), f32),
            pltpu.VMEM_SHARED((NPAD,), f32),
            pltpu.VMEM_SHARED((NPAD,), f32),
            pltpu.SemaphoreType.DMA,
            pltpu.SemaphoreType.DMA,
            pltpu.SemaphoreType.DMA,
            pltpu.SemaphoreType.DMA,
            pltpu.SemaphoreType.DMA,
            pltpu.SemaphoreType.DMA,
            pltpu.SemaphoreType.DMA,
            pltpu.SemaphoreType.DMA,
        ],
    )
    e, lv = fn(rows, cols, sub, al)
    E = jnp.stack([e[:N_NODES], e[NPAD:NPAD + N_NODES]], axis=1)
    return (lv[0, 0] + lv[1, 0], E)
